# baseline (speedup 1.0000x reference)
import functools

import jax
import jax.numpy as jnp
from jax import lax
from jax.experimental import pallas as pl
from jax.experimental.pallas import tpu as pltpu
from jax.experimental.pallas import tpu_sc as plsc

B = 1024
S = 100000
D = 64
C = 500
CP = 512
TEMP = 0.05
EPS = 1e-6

NC = 2
NS = 16
NW = NC * NS
BATCH = 128
NFULL = S // BATCH
TAIL = S - NFULL * BATCH
NB = -(-NFULL // NW)
LANES = 16


def _sc_segment_stats(featT, labels, indexes, zeros_f, zeros_c, ones_c):
  mesh = plsc.VectorSubcoreMesh(core_axis_name="c", subcore_axis_name="s")

  @functools.partial(
      pl.kernel,
      out_type=[
          jax.ShapeDtypeStruct((NC, CP, D), jnp.float32),
          jax.ShapeDtypeStruct((NC, CP, LANES), jnp.float32),
          jax.ShapeDtypeStruct((B,), jnp.int32),
      ],
      mesh=mesh,
      compiler_params=pltpu.CompilerParams(needs_layout_passes=False),
      scratch_types=[
          pltpu.VMEM((D, BATCH), jnp.float32),
          pltpu.VMEM((D, BATCH), jnp.float32),
          pltpu.VMEM((BATCH, D), jnp.float32),
          pltpu.VMEM((BATCH,), jnp.int32),
          pltpu.VMEM((BATCH,), jnp.int32),
          pltpu.VMEM((BATCH, LANES), jnp.float32),
          pltpu.VMEM((B // NW,), jnp.int32),
          pltpu.VMEM((B // NW,), jnp.int32),
          pltpu.SemaphoreType.DMA,
          pltpu.SemaphoreType.DMA,
          pltpu.SemaphoreType.DMA,
          pltpu.SemaphoreType.DMA,
          pltpu.SemaphoreType.DMA,
          pltpu.VMEM_SHARED((CP, D), jnp.float32),
          pltpu.VMEM_SHARED((CP, LANES), jnp.float32),
      ],
  )
  def k(featT_hbm, labf_hbm, idx_hbm, zf_hbm, zc_hbm, ones_hbm,
        fsum_out, cnt_out, tgt_out,
        featT_v0, featT_v1, rows_v, lab_v0, lab_v1, ones_v,
        idx_v, tgt_v, sem, fsem0, fsem1, lsem0, lsem1,
        fsum_sh, cnt_sh):
    cid = lax.axis_index("c")
    sid = lax.axis_index("s")
    wid = cid * NS + sid
    featT_vs, lab_vs = (featT_v0, featT_v1), (lab_v0, lab_v1)
    fsems, lsems = (fsem0, fsem1), (lsem0, lsem1)

    @pl.when(sid == 0)
    def _zero():
      pltpu.sync_copy(zf_hbm, fsum_sh)
      pltpu.sync_copy(zc_hbm, cnt_sh)

    plsc.subcore_barrier()

    pltpu.sync_copy(ones_hbm, ones_v)
    iotas = [lax.iota(jnp.int32, LANES) + kk * LANES
             for kk in range(BATCH // LANES)]

    def start(j):
      g = j * NW + wid

      @pl.when(g < NFULL)
      def _():
        off = pl.multiple_of(g * BATCH, BATCH)
        pltpu.async_copy(labf_hbm.at[pl.ds(off, BATCH)], lab_vs[j % 2],
                         lsems[j % 2])
        pltpu.async_copy(featT_hbm.at[:, pl.ds(off, BATCH)], featT_vs[j % 2],
                         fsems[j % 2])

    def finish(j):
      g = j * NW + wid

      @pl.when(g < NFULL)
      def _():
        off = pl.multiple_of(g * BATCH, BATCH)
        pltpu.make_async_copy(labf_hbm.at[pl.ds(off, BATCH)], lab_vs[j % 2],
                              lsems[j % 2]).wait()
        pltpu.make_async_copy(featT_hbm.at[:, pl.ds(off, BATCH)],
                              featT_vs[j % 2], fsems[j % 2]).wait()
        src = featT_vs[j % 2]

        def transp(d, carry):
          col = jnp.full((LANES,), d, jnp.int32)
          for kk in range(BATCH // LANES):
            v = src[d, pl.ds(kk * LANES, LANES)]
            plsc.store_scatter(rows_v, [iotas[kk], col], v)
          return carry

        lax.fori_loop(0, D, transp, 0)
        pltpu.sync_copy(rows_v, fsum_sh.at[lab_vs[j % 2]], add=True)
        pltpu.sync_copy(ones_v, cnt_sh.at[lab_vs[j % 2]], add=True)

    start(0)
    for j in range(NB):
      if j + 1 < NB:
        start(j + 1)
      finish(j)

    bpw = B // NW
    pltpu.sync_copy(idx_hbm.at[pl.ds(wid * bpw, bpw)], idx_v)
    pltpu.async_copy(labf_hbm.at[idx_v], tgt_v, sem).wait()
    pltpu.sync_copy(tgt_v, tgt_out.at[pl.ds(wid * bpw, bpw)])

    plsc.subcore_barrier()

    @pl.when(sid == 0)
    def _writeback():
      pltpu.sync_copy(fsum_sh, fsum_out.at[cid])
      pltpu.sync_copy(cnt_sh, cnt_out.at[cid])

  return k(featT, labels, indexes, zeros_f, zeros_c, ones_c)


def _tc_loss(inputs, fsum_parts, cnt_parts, targets2d, feat_tail, labels_tail):
  def body(x_ref, f_ref, c_ref, t_ref, ft_ref, lt_ref, loss_ref):
    x = x_ref[...]
    oh_tail = (lax.broadcasted_iota(jnp.int32, (CP, TAIL), 0)
               == lt_ref[...]).astype(jnp.float32)
    f_tail = lax.dot_general(oh_tail, ft_ref[...], (((1,), (0,)), ((), ())),
                             preferred_element_type=jnp.float32)
    f = f_ref[0] + f_ref[1] + f_tail
    cnt_tail = jnp.sum(oh_tail, axis=1, keepdims=True)
    cnt2 = c_ref[0] + c_ref[1] + cnt_tail
    cnt_col = cnt2[:, 0:1]
    denom = TEMP * jnp.where(cnt_col > 0, cnt_col, 1.0)
    fmean = f / denom
    sim = lax.dot_general(x, fmean, (((1,), (1,)), ((), ())),
                          preferred_element_type=jnp.float32)
    ones_row = jnp.ones((1, LANES), jnp.float32)
    cnt_row = lax.dot_general(ones_row, cnt2, (((1,), (1,)), ((), ())),
                              preferred_element_type=jnp.float32)
    maskr = (cnt_row > 0).astype(jnp.float32)
    exps = jnp.exp(sim) * maskr
    sums = jnp.sum(exps, axis=1, keepdims=True) + EPS
    logp = jnp.log(exps / sums + EPS)
    t = t_ref[...]
    cls = lax.broadcasted_iota(jnp.int32, (B, CP), 1)
    onehot = (cls == t).astype(jnp.float32)
    loss = -jnp.sum(onehot * logp) * (1.0 / B)
    loss_ref[...] = jnp.reshape(loss, (1, 1))

  return pl.pallas_call(
      body,
      out_shape=jax.ShapeDtypeStruct((1, 1), jnp.float32),
  )(inputs, fsum_parts, cnt_parts, targets2d, feat_tail, labels_tail)


def kernel(inputs, indexes, features, labels):
  zeros_f = jnp.zeros((CP, D), jnp.float32)
  zeros_c = jnp.zeros((CP, LANES), jnp.float32)
  ones_c = jnp.ones((BATCH, LANES), jnp.float32)
  featT = features.T
  tail_lo = NFULL * BATCH
  feat_tail = features[tail_lo:]
  labels_tail = labels[tail_lo:].reshape(1, TAIL)
  fsum, cnts, targets = _sc_segment_stats(
      featT, labels, indexes, zeros_f, zeros_c, ones_c)
  loss = _tc_loss(inputs, fsum, cnts, targets.reshape(B, 1),
                  feat_tail, labels_tail)
  return loss[0, 0]


# speedup vs baseline: 1.8051x; 1.8051x over previous
import functools

import jax
import jax.numpy as jnp
from jax import lax
from jax.experimental import pallas as pl
from jax.experimental.pallas import tpu as pltpu
from jax.experimental.pallas import tpu_sc as plsc

B = 1024
S = 100000
D = 64
C = 500
CP = 512
TEMP = 0.05
EPS = 1e-6

NC = 2
NS = 16
NW = NC * NS
BATCH = 128
NFULL = S // BATCH
TAIL = S - NFULL * BATCH
NB = -(-NFULL // NW)
LANES = 16


def _sc_segment_stats(features, labels, indexes, zeros_f, zeros_c, ones_c):
  mesh = plsc.VectorSubcoreMesh(core_axis_name="c", subcore_axis_name="s")

  @functools.partial(
      pl.kernel,
      out_type=[
          jax.ShapeDtypeStruct((NC, CP, D), jnp.float32),
          jax.ShapeDtypeStruct((NC, CP, LANES), jnp.float32),
          jax.ShapeDtypeStruct((B,), jnp.int32),
      ],
      mesh=mesh,
      compiler_params=pltpu.CompilerParams(needs_layout_passes=False),
      scratch_types=[
          pltpu.VMEM((BATCH, D), jnp.float32),
          pltpu.VMEM((BATCH, D), jnp.float32),
          pltpu.VMEM((BATCH,), jnp.int32),
          pltpu.VMEM((BATCH,), jnp.int32),
          pltpu.VMEM((TAIL, D), jnp.float32),
          pltpu.VMEM((TAIL,), jnp.int32),
          pltpu.VMEM((BATCH, LANES), jnp.float32),
          pltpu.VMEM((B // NW,), jnp.int32),
          pltpu.VMEM((B // NW,), jnp.int32),
          pltpu.SemaphoreType.DMA,
          pltpu.SemaphoreType.DMA,
          pltpu.SemaphoreType.DMA,
          pltpu.SemaphoreType.DMA,
          pltpu.SemaphoreType.DMA,
          pltpu.SemaphoreType.DMA,
          pltpu.SemaphoreType.DMA,
          pltpu.SemaphoreType.DMA,
          pltpu.SemaphoreType.DMA,
          pltpu.VMEM_SHARED((CP, D), jnp.float32),
          pltpu.VMEM_SHARED((CP, LANES), jnp.float32),
      ],
  )
  def k(feat_hbm, labf_hbm, idx_hbm, zf_hbm, zc_hbm, ones_hbm,
        fsum_out, cnt_out, tgt_out,
        feat_v0, feat_v1, lab_v0, lab_v1, feat_t, lab_t, ones_v,
        idx_v, tgt_v, sem, fsem0, fsem1, lsem0, lsem1,
        ssem0, ssem1, csem0, csem1,
        fsum_sh, cnt_sh):
    cid = lax.axis_index("c")
    sid = lax.axis_index("s")
    wid = cid * NS + sid
    feat_vs, lab_vs = (feat_v0, feat_v1), (lab_v0, lab_v1)
    fsems, lsems = (fsem0, fsem1), (lsem0, lsem1)
    ssems, csems = (ssem0, ssem1), (csem0, csem1)

    @pl.when(sid == 0)
    def _zero():
      pltpu.sync_copy(zf_hbm, fsum_sh)
      pltpu.sync_copy(zc_hbm, cnt_sh)

    plsc.subcore_barrier()

    pltpu.sync_copy(ones_hbm, ones_v)

    def start(j):
      g = j * NW + wid

      @pl.when(g < NFULL)
      def _():
        off = pl.multiple_of(g * BATCH, BATCH)
        pltpu.async_copy(labf_hbm.at[pl.ds(off, BATCH)], lab_vs[j % 2],
                         lsems[j % 2])
        pltpu.async_copy(feat_hbm.at[pl.ds(off, BATCH)], feat_vs[j % 2],
                         fsems[j % 2])

    def finish(j):
      g = j * NW + wid

      @pl.when(g < NFULL)
      def _():
        off = pl.multiple_of(g * BATCH, BATCH)
        pltpu.make_async_copy(labf_hbm.at[pl.ds(off, BATCH)], lab_vs[j % 2],
                              lsems[j % 2]).wait()
        pltpu.make_async_copy(feat_hbm.at[pl.ds(off, BATCH)], feat_vs[j % 2],
                              fsems[j % 2]).wait()
        pltpu.async_copy(feat_vs[j % 2], fsum_sh.at[lab_vs[j % 2]],
                         ssems[j % 2], add=True)
        pltpu.async_copy(ones_v, cnt_sh.at[lab_vs[j % 2]],
                         csems[j % 2], add=True)

    def wait_scatter(j):
      g = j * NW + wid

      @pl.when(g < NFULL)
      def _():
        pltpu.make_async_copy(feat_vs[j % 2], fsum_sh.at[lab_vs[j % 2]],
                              ssems[j % 2]).wait()
        pltpu.make_async_copy(ones_v, cnt_sh.at[lab_vs[j % 2]],
                              csems[j % 2]).wait()

    start(0)
    for j in range(NB):
      if j + 1 < NB:
        if j >= 1:
          wait_scatter(j - 1)
        start(j + 1)
      finish(j)
    wait_scatter(NB - 2)
    wait_scatter(NB - 1)

    @pl.when(wid == NW - 1)
    def _tail():
      pltpu.sync_copy(labf_hbm.at[pl.ds(NFULL * BATCH, TAIL)], lab_t)
      pltpu.sync_copy(feat_hbm.at[pl.ds(NFULL * BATCH, TAIL)], feat_t)
      pltpu.sync_copy(feat_t, fsum_sh.at[lab_t], add=True)
      pltpu.sync_copy(ones_v.at[pl.ds(0, TAIL)], cnt_sh.at[lab_t], add=True)

    bpw = B // NW
    pltpu.sync_copy(idx_hbm.at[pl.ds(wid * bpw, bpw)], idx_v)
    pltpu.async_copy(labf_hbm.at[idx_v], tgt_v, sem).wait()
    pltpu.sync_copy(tgt_v, tgt_out.at[pl.ds(wid * bpw, bpw)])

    plsc.subcore_barrier()

    @pl.when(sid == 0)
    def _writeback():
      pltpu.sync_copy(fsum_sh, fsum_out.at[cid])
      pltpu.sync_copy(cnt_sh, cnt_out.at[cid])

  return k(features, labels, indexes, zeros_f, zeros_c, ones_c)


def _tc_loss(inputs, fsum_parts, cnt_parts, targets2d):
  def body(x_ref, f_ref, c_ref, t_ref, loss_ref):
    x = x_ref[...]
    f = f_ref[0] + f_ref[1]
    cnt2 = c_ref[0] + c_ref[1]
    cnt_col = cnt2[:, 0:1]
    denom = TEMP * jnp.where(cnt_col > 0, cnt_col, 1.0)
    fmean = f / denom
    sim = lax.dot_general(x, fmean, (((1,), (1,)), ((), ())),
                          preferred_element_type=jnp.float32)
    ones_row = jnp.ones((1, LANES), jnp.float32)
    cnt_row = lax.dot_general(ones_row, cnt2, (((1,), (1,)), ((), ())),
                              preferred_element_type=jnp.float32)
    maskr = (cnt_row > 0).astype(jnp.float32)
    exps = jnp.exp(sim) * maskr
    sums = jnp.sum(exps, axis=1, keepdims=True) + EPS
    logp = jnp.log(exps / sums + EPS)
    t = t_ref[...]
    cls = lax.broadcasted_iota(jnp.int32, (B, CP), 1)
    onehot = (cls == t).astype(jnp.float32)
    loss = -jnp.sum(onehot * logp) * (1.0 / B)
    loss_ref[...] = jnp.reshape(loss, (1, 1))

  return pl.pallas_call(
      body,
      out_shape=jax.ShapeDtypeStruct((1, 1), jnp.float32),
  )(inputs, fsum_parts, cnt_parts, targets2d)


def kernel(inputs, indexes, features, labels):
  zeros_f = jnp.zeros((CP, D), jnp.float32)
  zeros_c = jnp.zeros((CP, LANES), jnp.float32)
  ones_c = jnp.ones((BATCH, LANES), jnp.float32)
  fsum, cnts, targets = _sc_segment_stats(
      features, labels, indexes, zeros_f, zeros_c, ones_c)
  loss = _tc_loss(inputs, fsum, cnts, targets.reshape(B, 1))
  return loss[0, 0]


# speedup vs baseline: 1.8228x; 1.0098x over previous
import functools

import jax
import jax.numpy as jnp
from jax import lax
from jax.experimental import pallas as pl
from jax.experimental.pallas import tpu as pltpu
from jax.experimental.pallas import tpu_sc as plsc

B = 1024
S = 100000
D = 64
C = 500
CP = 512
TEMP = 0.05
EPS = 1e-6

NC = 2
NS = 16
NW = NC * NS
BATCH = 128
NFULL = S // BATCH
TAIL = S - NFULL * BATCH
NB = -(-NFULL // NW)
LANES = 16


def _sc_segment_stats(features, labels, indexes, zeros_f, zeros_c, ones_c):
  mesh = plsc.VectorSubcoreMesh(core_axis_name="c", subcore_axis_name="s")

  @functools.partial(
      pl.kernel,
      out_type=[
          jax.ShapeDtypeStruct((NC, CP, D), jnp.float32),
          jax.ShapeDtypeStruct((NC, CP, LANES), jnp.float32),
          jax.ShapeDtypeStruct((B,), jnp.int32),
      ],
      mesh=mesh,
      compiler_params=pltpu.CompilerParams(needs_layout_passes=False),
      scratch_types=[
          pltpu.VMEM((BATCH, D), jnp.float32),
          pltpu.VMEM((BATCH, D), jnp.float32),
          pltpu.VMEM((BATCH,), jnp.int32),
          pltpu.VMEM((BATCH,), jnp.int32),
          pltpu.VMEM((TAIL, D), jnp.float32),
          pltpu.VMEM((TAIL,), jnp.int32),
          pltpu.VMEM((BATCH, LANES), jnp.float32),
          pltpu.VMEM((B // NW,), jnp.int32),
          pltpu.VMEM((B // NW,), jnp.int32),
          pltpu.SemaphoreType.DMA,
          pltpu.SemaphoreType.DMA,
          pltpu.SemaphoreType.DMA,
          pltpu.SemaphoreType.DMA,
          pltpu.SemaphoreType.DMA,
          pltpu.VMEM_SHARED((CP, D), jnp.float32),
          pltpu.VMEM_SHARED((CP, LANES), jnp.float32),
      ],
  )
  def k(feat_hbm, labf_hbm, idx_hbm, zf_hbm, zc_hbm, ones_hbm,
        fsum_out, cnt_out, tgt_out,
        feat_v0, feat_v1, lab_v0, lab_v1, feat_t, lab_t, ones_v,
        idx_v, tgt_v, sem, fsem0, fsem1, lsem0, lsem1,
        fsum_sh, cnt_sh):
    cid = lax.axis_index("c")
    sid = lax.axis_index("s")
    wid = cid * NS + sid
    feat_vs, lab_vs = (feat_v0, feat_v1), (lab_v0, lab_v1)
    fsems, lsems = (fsem0, fsem1), (lsem0, lsem1)

    @pl.when(sid == 0)
    def _zero():
      pltpu.sync_copy(zf_hbm, fsum_sh)
      pltpu.sync_copy(zc_hbm, cnt_sh)

    plsc.subcore_barrier()

    pltpu.sync_copy(ones_hbm, ones_v)

    def start(j):
      g = j * NW + wid

      @pl.when(g < NFULL)
      def _():
        off = pl.multiple_of(g * BATCH, BATCH)
        pltpu.async_copy(labf_hbm.at[pl.ds(off, BATCH)], lab_vs[j % 2],
                         lsems[j % 2])
        pltpu.async_copy(feat_hbm.at[pl.ds(off, BATCH)], feat_vs[j % 2],
                         fsems[j % 2])

    def finish(j):
      g = j * NW + wid

      @pl.when(g < NFULL)
      def _():
        off = pl.multiple_of(g * BATCH, BATCH)
        pltpu.make_async_copy(labf_hbm.at[pl.ds(off, BATCH)], lab_vs[j % 2],
                              lsems[j % 2]).wait()
        pltpu.make_async_copy(feat_hbm.at[pl.ds(off, BATCH)], feat_vs[j % 2],
                              fsems[j % 2]).wait()
        pltpu.sync_copy(feat_vs[j % 2], fsum_sh.at[lab_vs[j % 2]], add=True)
        pltpu.sync_copy(ones_v, cnt_sh.at[lab_vs[j % 2]], add=True)

    start(0)
    for j in range(NB):
      if j + 1 < NB:
        start(j + 1)
      finish(j)

    @pl.when(wid == NW - 1)
    def _tail():
      pltpu.sync_copy(labf_hbm.at[pl.ds(NFULL * BATCH, TAIL)], lab_t)
      pltpu.sync_copy(feat_hbm.at[pl.ds(NFULL * BATCH, TAIL)], feat_t)
      pltpu.sync_copy(feat_t, fsum_sh.at[lab_t], add=True)
      pltpu.sync_copy(ones_v.at[pl.ds(0, TAIL)], cnt_sh.at[lab_t], add=True)

    bpw = B // NW
    pltpu.sync_copy(idx_hbm.at[pl.ds(wid * bpw, bpw)], idx_v)
    pltpu.async_copy(labf_hbm.at[idx_v], tgt_v, sem).wait()
    pltpu.sync_copy(tgt_v, tgt_out.at[pl.ds(wid * bpw, bpw)])

    plsc.subcore_barrier()

    @pl.when(sid == 0)
    def _writeback():
      pltpu.sync_copy(fsum_sh, fsum_out.at[cid])
      pltpu.sync_copy(cnt_sh, cnt_out.at[cid])

  return k(features, labels, indexes, zeros_f, zeros_c, ones_c)


def _tc_loss(inputs, fsum_parts, cnt_parts, targets2d):
  def body(x_ref, f_ref, c_ref, t_ref, loss_ref):
    x = x_ref[...]
    f = f_ref[0] + f_ref[1]
    cnt2 = c_ref[0] + c_ref[1]
    cnt_col = cnt2[:, 0:1]
    denom = TEMP * jnp.where(cnt_col > 0, cnt_col, 1.0)
    fmean = f / denom
    sim = lax.dot_general(x, fmean, (((1,), (1,)), ((), ())),
                          preferred_element_type=jnp.float32)
    ones_row = jnp.ones((1, LANES), jnp.float32)
    cnt_row = lax.dot_general(ones_row, cnt2, (((1,), (1,)), ((), ())),
                              preferred_element_type=jnp.float32)
    maskr = (cnt_row > 0).astype(jnp.float32)
    exps = jnp.exp(sim) * maskr
    sums = jnp.sum(exps, axis=1, keepdims=True) + EPS
    logp = jnp.log(exps / sums + EPS)
    t = t_ref[...]
    cls = lax.broadcasted_iota(jnp.int32, (B, CP), 1)
    onehot = (cls == t).astype(jnp.float32)
    loss = -jnp.sum(onehot * logp) * (1.0 / B)
    loss_ref[...] = jnp.reshape(loss, (1, 1))

  return pl.pallas_call(
      body,
      out_shape=jax.ShapeDtypeStruct((1, 1), jnp.float32),
  )(inputs, fsum_parts, cnt_parts, targets2d)


def kernel(inputs, indexes, features, labels):
  zeros_f = jnp.zeros((CP, D), jnp.float32)
  zeros_c = jnp.zeros((CP, LANES), jnp.float32)
  ones_c = jnp.ones((BATCH, LANES), jnp.float32)
  fsum, cnts, targets = _sc_segment_stats(
      features, labels, indexes, zeros_f, zeros_c, ones_c)
  loss = _tc_loss(inputs, fsum, cnts, targets.reshape(B, 1))
  return loss[0, 0]
